# Initial kernel scaffold; baseline (speedup 1.0000x reference)
import jax
import jax.numpy as jnp
from jax.experimental import pallas as pl


def kernel(z, edge_index, edge_type, weight):
    raise NotImplementedError("write your pallas kernel here")


# speedup vs baseline: 3.3553x; 3.3553x over previous
import functools

import jax
import jax.numpy as jnp
from jax import lax
from jax.experimental import pallas as pl
from jax.experimental.pallas import tpu as pltpu
from jax.experimental.pallas import tpu_sc as plsc

D = 128
LANES = 16
NW = 32
C = 80


def _sc_decode(z, src_idx, dst_idx, rel_idx, weight, n_edges):
    epw = n_edges // NW
    n_chunks = epw // C

    mesh = plsc.VectorSubcoreMesh(core_axis_name="c", subcore_axis_name="s")

    @functools.partial(
        pl.kernel,
        out_type=jax.ShapeDtypeStruct((n_edges,), jnp.float32),
        mesh=mesh,
        compiler_params=pltpu.CompilerParams(needs_layout_passes=False),
        scratch_types=[
            pltpu.VMEM((C,), jnp.int32),
            pltpu.VMEM((C,), jnp.int32),
            pltpu.VMEM((C,), jnp.int32),
            pltpu.VMEM((C, D), jnp.float32),
            pltpu.VMEM((C, D), jnp.float32),
            pltpu.VMEM((C, D), jnp.float32),
            pltpu.VMEM((C,), jnp.float32),
            pltpu.SemaphoreType.DMA,
        ],
    )
    def decode(z_hbm, src_hbm, dst_hbm, rel_hbm, w_hbm, out_hbm,
               si_v, di_v, ri_v, sr_v, dr_v, rr_v, ob_v, sem):
        wid = lax.axis_index("s") * 2 + lax.axis_index("c")
        base0 = wid * epw
        lane15 = lax.iota(jnp.int32, 16) == 15

        def chunk_body(g, carry):
            base = base0 + g * C
            pltpu.sync_copy(src_hbm.at[pl.ds(base, C)], si_v)
            pltpu.sync_copy(dst_hbm.at[pl.ds(base, C)], di_v)
            pltpu.sync_copy(rel_hbm.at[pl.ds(base, C)], ri_v)
            cp_s = pltpu.async_copy(z_hbm.at[si_v], sr_v, sem)
            cp_d = pltpu.async_copy(z_hbm.at[di_v], dr_v, sem)
            cp_r = pltpu.async_copy(w_hbm.at[ri_v], rr_v, sem)
            cp_s.wait()
            cp_d.wait()
            cp_r.wait()

            def edge_body(e, carry2):
                acc = (sr_v[e, pl.ds(0, LANES)]
                       * dr_v[e, pl.ds(0, LANES)]
                       * rr_v[e, pl.ds(0, LANES)])
                for j in range(1, D // LANES):
                    acc = acc + (sr_v[e, pl.ds(j * LANES, LANES)]
                                 * dr_v[e, pl.ds(j * LANES, LANES)]
                                 * rr_v[e, pl.ds(j * LANES, LANES)])
                csum = plsc.cumsum(acc)
                idxv = jnp.full((LANES,), e, dtype=jnp.int32)
                plsc.store_scatter(ob_v, [idxv], csum, mask=lane15)
                return carry2

            lax.fori_loop(0, C, edge_body, 0, unroll=2)

            for i in range(C // LANES):
                v = ob_v[pl.ds(i * LANES, LANES)]
                ob_v[pl.ds(i * LANES, LANES)] = 1.0 / (1.0 + jnp.exp(-v))
            pltpu.sync_copy(ob_v, out_hbm.at[pl.ds(base, C)])
            return carry

        lax.fori_loop(0, n_chunks, chunk_body, 0)

    return decode(z, src_idx, dst_idx, rel_idx, weight)


def kernel(z, edge_index, edge_type, weight):
    n_edges = edge_index.shape[1]
    src_idx = edge_index[0]
    dst_idx = edge_index[1]
    return _sc_decode(z, src_idx, dst_idx, edge_type, weight, n_edges)


# speedup vs baseline: 7.4707x; 2.2265x over previous
import functools

import jax
import jax.numpy as jnp
from jax import lax
from jax.experimental import pallas as pl
from jax.experimental.pallas import tpu as pltpu
from jax.experimental.pallas import tpu_sc as plsc

D = 128
LANES = 16
NW = 32
C = 80


def _sc_decode(z, src_idx, dst_idx, rel_idx, weight, n_edges):
    epw = n_edges // NW
    n_chunks = epw // C
    n_pairs = (n_chunks - 1) // 2

    mesh = plsc.VectorSubcoreMesh(core_axis_name="c", subcore_axis_name="s")

    @functools.partial(
        pl.kernel,
        out_type=jax.ShapeDtypeStruct((n_edges,), jnp.float32),
        mesh=mesh,
        compiler_params=pltpu.CompilerParams(needs_layout_passes=False),
        scratch_types=[
            pltpu.VMEM((epw,), jnp.int32),
            pltpu.VMEM((epw,), jnp.int32),
            pltpu.VMEM((epw,), jnp.int32),
            pltpu.VMEM((C, D), jnp.float32),
            pltpu.VMEM((C, D), jnp.float32),
            pltpu.VMEM((C, D), jnp.float32),
            pltpu.VMEM((C, D), jnp.float32),
            pltpu.VMEM((C, D), jnp.float32),
            pltpu.VMEM((C, D), jnp.float32),
            pltpu.VMEM((LANES, LANES), jnp.float32),
            pltpu.VMEM((C,), jnp.float32),
            pltpu.SemaphoreType.DMA,
            pltpu.SemaphoreType.DMA,
        ],
    )
    def decode(z_hbm, src_hbm, dst_hbm, rel_hbm, w_hbm, out_hbm,
               si_v, di_v, ri_v, sra, dra, rra, srb, drb, rrb,
               t_v, ob_v, sem_a, sem_b):
        wid = lax.axis_index("s") * 2 + lax.axis_index("c")
        base0 = wid * epw
        iota = lax.iota(jnp.int32, LANES)

        def row_copies(g, sr, dr, rr, sem):
            off = g * C
            return (
                pltpu.make_async_copy(z_hbm.at[si_v.at[pl.ds(off, C)]], sr, sem),
                pltpu.make_async_copy(z_hbm.at[di_v.at[pl.ds(off, C)]], dr, sem),
                pltpu.make_async_copy(w_hbm.at[ri_v.at[pl.ds(off, C)]], rr, sem),
            )

        def issue(g, sr, dr, rr, sem):
            for cp in row_copies(g, sr, dr, rr, sem):
                cp.start()

        def wait(g, sr, dr, rr, sem):
            for cp in row_copies(g, sr, dr, rr, sem):
                cp.wait()

        def score_chunk(g, sr, dr, rr):
            def group_body(grp, carry):
                gb = grp * LANES

                def edge_body(k, carry2):
                    acc = (sr[gb + k, pl.ds(0, LANES)]
                           * dr[gb + k, pl.ds(0, LANES)]
                           * rr[gb + k, pl.ds(0, LANES)])
                    for j in range(1, D // LANES):
                        acc = acc + (sr[gb + k, pl.ds(j * LANES, LANES)]
                                     * dr[gb + k, pl.ds(j * LANES, LANES)]
                                     * rr[gb + k, pl.ds(j * LANES, LANES)])
                    t_v[k, :] = acc
                    return carry2

                lax.fori_loop(0, LANES, edge_body, 0, unroll=4)

                s = plsc.load_gather(t_v, [iota, jnp.zeros((LANES,), jnp.int32)])
                for k in range(1, LANES):
                    s = s + plsc.load_gather(
                        t_v, [iota, jnp.full((LANES,), k, jnp.int32)])
                s = 1.0 / (1.0 + jnp.exp(-s))
                ob_v[pl.ds(gb, LANES)] = s
                return carry

            lax.fori_loop(0, C // LANES, group_body, 0)
            pltpu.sync_copy(ob_v, out_hbm.at[pl.ds(base0 + g * C, C)])

        pltpu.sync_copy(src_hbm.at[pl.ds(base0, epw)], si_v)
        pltpu.sync_copy(dst_hbm.at[pl.ds(base0, epw)], di_v)
        pltpu.sync_copy(rel_hbm.at[pl.ds(base0, epw)], ri_v)

        issue(0, sra, dra, rra, sem_a)

        def pair_body(i, carry):
            ga = 2 * i
            issue(ga + 1, srb, drb, rrb, sem_b)
            wait(ga, sra, dra, rra, sem_a)
            score_chunk(ga, sra, dra, rra)
            issue(ga + 2, sra, dra, rra, sem_a)
            wait(ga + 1, srb, drb, rrb, sem_b)
            score_chunk(ga + 1, srb, drb, rrb)
            return carry

        lax.fori_loop(0, n_pairs, pair_body, 0)
        last = n_chunks - 1
        wait(last, sra, dra, rra, sem_a)
        score_chunk(last, sra, dra, rra)

    return decode(z, src_idx, dst_idx, rel_idx, weight)


def kernel(z, edge_index, edge_type, weight):
    n_edges = edge_index.shape[1]
    src_idx = edge_index[0]
    dst_idx = edge_index[1]
    return _sc_decode(z, src_idx, dst_idx, edge_type, weight, n_edges)


# speedup vs baseline: 8.0628x; 1.0793x over previous
import functools

import jax
import jax.numpy as jnp
from jax import lax
from jax.experimental import pallas as pl
from jax.experimental.pallas import tpu as pltpu
from jax.experimental.pallas import tpu_sc as plsc

D = 128
DW = D // 2
LANES = 16
NW = 32
C = 80


def _pack_bf16(a):
    n = a.shape[0]
    return lax.bitcast_convert_type(
        a.astype(jnp.bfloat16).reshape(n, DW, 2), jnp.float32)


def _sc_decode(zp, src_idx, dst_idx, rel_idx, wp, n_edges):
    epw = n_edges // NW
    n_chunks = epw // C
    n_pairs = (n_chunks - 1) // 2
    num_et = wp.shape[0]

    mesh = plsc.VectorSubcoreMesh(core_axis_name="c", subcore_axis_name="s")

    @functools.partial(
        pl.kernel,
        out_type=jax.ShapeDtypeStruct((n_edges,), jnp.float32),
        mesh=mesh,
        compiler_params=pltpu.CompilerParams(needs_layout_passes=False,
                                             use_tc_tiling_on_sc=False),
        scratch_types=[
            pltpu.VMEM((epw,), jnp.int32),
            pltpu.VMEM((epw,), jnp.int32),
            pltpu.VMEM((epw,), jnp.int32),
            pltpu.VMEM((C, DW), jnp.float32),
            pltpu.VMEM((C, DW), jnp.float32),
            pltpu.VMEM((C, DW), jnp.float32),
            pltpu.VMEM((C, DW), jnp.float32),
            pltpu.VMEM((num_et, DW), jnp.float32),
            pltpu.VMEM((LANES, LANES), jnp.float32),
            pltpu.VMEM((C,), jnp.float32),
            pltpu.SemaphoreType.DMA,
            pltpu.SemaphoreType.DMA,
        ],
    )
    def decode(z_hbm, src_hbm, dst_hbm, rel_hbm, w_hbm, out_hbm,
               si_v, di_v, ri_v, sra, dra, srb, drb, w_t,
               t_v, ob_v, sem_a, sem_b):
        wid = lax.axis_index("s") * 2 + lax.axis_index("c")
        base0 = wid * epw
        iota = lax.iota(jnp.int32, LANES)

        pltpu.sync_copy(w_hbm, w_t)
        pltpu.sync_copy(src_hbm.at[pl.ds(base0, epw)], si_v)
        pltpu.sync_copy(dst_hbm.at[pl.ds(base0, epw)], di_v)
        pltpu.sync_copy(rel_hbm.at[pl.ds(base0, epw)], ri_v)

        def row_copies(g, sr, dr, sem):
            off = g * C
            return (
                pltpu.make_async_copy(z_hbm.at[si_v.at[pl.ds(off, C)]], sr, sem),
                pltpu.make_async_copy(z_hbm.at[di_v.at[pl.ds(off, C)]], dr, sem),
            )

        def issue(g, sr, dr, sem):
            for cp in row_copies(g, sr, dr, sem):
                cp.start()

        def wait(g, sr, dr, sem):
            for cp in row_copies(g, sr, dr, sem):
                cp.wait()

        def mul2(sv, dv, wv):
            s0, s1 = plsc.unpack(plsc.bitcast(sv, jnp.bfloat16), format=plsc.PackFormat.INTERLEAVED)
            d0, d1 = plsc.unpack(plsc.bitcast(dv, jnp.bfloat16), format=plsc.PackFormat.INTERLEAVED)
            w0, w1 = plsc.unpack(plsc.bitcast(wv, jnp.bfloat16), format=plsc.PackFormat.INTERLEAVED)
            return s0 * d0 * w0 + s1 * d1 * w1

        def score_chunk(g, sr, dr):
            def group_body(grp, carry):
                gb = grp * LANES
                relv = ri_v[pl.ds(g * C + gb, LANES)]

                for k in range(LANES):
                    rel = relv[k]
                    acc = mul2(sr[gb + k, pl.ds(0, LANES)],
                               dr[gb + k, pl.ds(0, LANES)],
                               w_t[rel, pl.ds(0, LANES)])
                    for j in range(1, DW // LANES):
                        acc = acc + mul2(sr[gb + k, pl.ds(j * LANES, LANES)],
                                         dr[gb + k, pl.ds(j * LANES, LANES)],
                                         w_t[rel, pl.ds(j * LANES, LANES)])
                    t_v[k, :] = acc

                s = plsc.load_gather(t_v, [iota, jnp.zeros((LANES,), jnp.int32)])
                for k in range(1, LANES):
                    s = s + plsc.load_gather(
                        t_v, [iota, jnp.full((LANES,), k, jnp.int32)])
                s = 1.0 / (1.0 + jnp.exp(-s))
                ob_v[pl.ds(gb, LANES)] = s
                return carry

            lax.fori_loop(0, C // LANES, group_body, 0)
            pltpu.sync_copy(ob_v, out_hbm.at[pl.ds(base0 + g * C, C)])

        issue(0, sra, dra, sem_a)

        def pair_body(i, carry):
            ga = 2 * i
            issue(ga + 1, srb, drb, sem_b)
            wait(ga, sra, dra, sem_a)
            score_chunk(ga, sra, dra)
            issue(ga + 2, sra, dra, sem_a)
            wait(ga + 1, srb, drb, sem_b)
            score_chunk(ga + 1, srb, drb)
            return carry

        lax.fori_loop(0, n_pairs, pair_body, 0)
        last = n_chunks - 1
        wait(last, sra, dra, sem_a)
        score_chunk(last, sra, dra)

    return decode(zp, src_idx, dst_idx, rel_idx, wp)


def kernel(z, edge_index, edge_type, weight):
    n_edges = edge_index.shape[1]
    src_idx = edge_index[0]
    dst_idx = edge_index[1]
    return _sc_decode(_pack_bf16(z), src_idx, dst_idx, edge_type,
                      _pack_bf16(weight), n_edges)


# speedup vs baseline: 8.1118x; 1.0061x over previous
import functools

import jax
import jax.numpy as jnp
from jax import lax
from jax.experimental import pallas as pl
from jax.experimental.pallas import tpu as pltpu
from jax.experimental.pallas import tpu_sc as plsc

D = 128
DW = D // 2
LANES = 16
NW = 32
C = 80


def _pack_bf16(a):
    n = a.shape[0]
    return lax.bitcast_convert_type(
        a.astype(jnp.bfloat16).reshape(n, DW, 2), jnp.float32)


def _sc_decode(zp, src_idx, dst_idx, rel_idx, wp, n_edges):
    epw = n_edges // NW
    n_chunks = epw // C
    n_pairs = (n_chunks - 1) // 2
    num_et = wp.shape[0]

    mesh = plsc.VectorSubcoreMesh(core_axis_name="c", subcore_axis_name="s")

    @functools.partial(
        pl.kernel,
        out_type=jax.ShapeDtypeStruct((n_edges,), jnp.float32),
        mesh=mesh,
        compiler_params=pltpu.CompilerParams(needs_layout_passes=False,
                                             use_tc_tiling_on_sc=False),
        scratch_types=[
            pltpu.VMEM((epw,), jnp.int32),
            pltpu.VMEM((epw,), jnp.int32),
            pltpu.VMEM((epw,), jnp.int32),
            pltpu.VMEM((C, DW), jnp.float32),
            pltpu.VMEM((C, DW), jnp.float32),
            pltpu.VMEM((C, DW), jnp.float32),
            pltpu.VMEM((C, DW), jnp.float32),
            pltpu.VMEM((C, DW), jnp.float32),
            pltpu.VMEM((C, DW), jnp.float32),
            pltpu.VMEM((num_et, DW), jnp.float32),
            pltpu.VMEM((LANES, LANES), jnp.float32),
            pltpu.VMEM((C,), jnp.float32),
            pltpu.SemaphoreType.DMA,
            pltpu.SemaphoreType.DMA,
            pltpu.SemaphoreType.DMA,
        ],
    )
    def decode(z_hbm, src_hbm, dst_hbm, rel_hbm, w_hbm, out_hbm,
               si_v, di_v, ri_v, sr0, dr0, sr1, dr1, sr2, dr2, w_t,
               t_v, ob_v, sem0, sem1, sem2):
        wid = lax.axis_index("s") * 2 + lax.axis_index("c")
        base0 = wid * epw
        iota = lax.iota(jnp.int32, LANES)

        pltpu.sync_copy(w_hbm, w_t)
        pltpu.sync_copy(src_hbm.at[pl.ds(base0, epw)], si_v)
        pltpu.sync_copy(dst_hbm.at[pl.ds(base0, epw)], di_v)
        pltpu.sync_copy(rel_hbm.at[pl.ds(base0, epw)], ri_v)

        def row_copies(g, sr, dr, sem):
            off = g * C
            return (
                pltpu.make_async_copy(z_hbm.at[si_v.at[pl.ds(off, C)]], sr, sem),
                pltpu.make_async_copy(z_hbm.at[di_v.at[pl.ds(off, C)]], dr, sem),
            )

        def issue(g, sr, dr, sem):
            for cp in row_copies(g, sr, dr, sem):
                cp.start()

        def wait(g, sr, dr, sem):
            for cp in row_copies(g, sr, dr, sem):
                cp.wait()

        def mul2(sv, dv, wv):
            s0, s1 = plsc.unpack(plsc.bitcast(sv, jnp.bfloat16), format=plsc.PackFormat.INTERLEAVED)
            d0, d1 = plsc.unpack(plsc.bitcast(dv, jnp.bfloat16), format=plsc.PackFormat.INTERLEAVED)
            w0, w1 = plsc.unpack(plsc.bitcast(wv, jnp.bfloat16), format=plsc.PackFormat.INTERLEAVED)
            return s0 * d0 * w0 + s1 * d1 * w1

        def score_chunk(g, sr, dr):
            def group_body(grp, carry):
                gb = grp * LANES
                relv = ri_v[pl.ds(g * C + gb, LANES)]

                for k in range(LANES):
                    rel = relv[k]
                    acc = mul2(sr[gb + k, pl.ds(0, LANES)],
                               dr[gb + k, pl.ds(0, LANES)],
                               w_t[rel, pl.ds(0, LANES)])
                    for j in range(1, DW // LANES):
                        acc = acc + mul2(sr[gb + k, pl.ds(j * LANES, LANES)],
                                         dr[gb + k, pl.ds(j * LANES, LANES)],
                                         w_t[rel, pl.ds(j * LANES, LANES)])
                    t_v[k, :] = acc

                cols = [plsc.load_gather(
                            t_v, [iota, jnp.full((LANES,), k, jnp.int32)])
                        for k in range(LANES)]
                while len(cols) > 1:
                    cols = [a + b for a, b in zip(cols[0::2], cols[1::2])]
                s = cols[0]
                s = 1.0 / (1.0 + jnp.exp(-s))
                ob_v[pl.ds(gb, LANES)] = s
                return carry

            lax.fori_loop(0, C // LANES, group_body, 0)
            pltpu.sync_copy(ob_v, out_hbm.at[pl.ds(base0 + g * C, C)])

        bufs = ((sr0, dr0, sem0), (sr1, dr1, sem1), (sr2, dr2, sem2))
        issue(0, *bufs[0])
        issue(1, *bufs[1])

        def triple_body(i, carry):
            g = 3 * i
            for p in range(3):
                issue(g + p + 2, *bufs[(p + 2) % 3])
                wait(g + p, *bufs[p])
                score_chunk(g + p, bufs[p][0], bufs[p][1])
            return carry

        lax.fori_loop(0, (n_chunks - 2) // 3, triple_body, 0)
        for g in range(n_chunks - 2, n_chunks):
            b = bufs[g % 3]
            wait(g, *b)
            score_chunk(g, b[0], b[1])

    return decode(zp, src_idx, dst_idx, rel_idx, wp)


def kernel(z, edge_index, edge_type, weight):
    n_edges = edge_index.shape[1]
    src_idx = edge_index[0]
    dst_idx = edge_index[1]
    return _sc_decode(_pack_bf16(z), src_idx, dst_idx, edge_type,
                      _pack_bf16(weight), n_edges)


# speedup vs baseline: 8.7533x; 1.0791x over previous
import functools

import jax
import jax.numpy as jnp
from jax import lax
from jax.experimental import pallas as pl
from jax.experimental.pallas import tpu as pltpu
from jax.experimental.pallas import tpu_sc as plsc

D = 128
DW = D // 2
LANES = 16
NW = 32
C = 80


def _pack_bf16(a):
    n = a.shape[0]
    return lax.bitcast_convert_type(
        a.astype(jnp.bfloat16).reshape(n, DW, 2), jnp.float32)


def _sc_decode(zp, src_idx, dst_idx, rel_idx, wp, n_edges):
    epw = n_edges // NW
    n_chunks = epw // C

    mesh = plsc.VectorSubcoreMesh(core_axis_name="c", subcore_axis_name="s")

    @functools.partial(
        pl.kernel,
        out_type=jax.ShapeDtypeStruct((n_edges,), jnp.float32),
        mesh=mesh,
        compiler_params=pltpu.CompilerParams(needs_layout_passes=False,
                                             use_tc_tiling_on_sc=False),
        scratch_types=[
            pltpu.VMEM((epw,), jnp.int32),
            pltpu.VMEM((epw,), jnp.int32),
            pltpu.VMEM((epw,), jnp.int32),
            pltpu.VMEM((C, DW), jnp.float32),
            pltpu.VMEM((C, DW), jnp.float32),
            pltpu.VMEM((C, DW), jnp.float32),
            pltpu.VMEM((C, DW), jnp.float32),
            pltpu.VMEM((C, DW), jnp.float32),
            pltpu.VMEM((C, DW), jnp.float32),
            pltpu.VMEM((LANES, LANES), jnp.float32),
            pltpu.VMEM((C,), jnp.float32),
            pltpu.VMEM((C,), jnp.float32),
            pltpu.SemaphoreType.DMA,
            pltpu.SemaphoreType.DMA,
            pltpu.SemaphoreType.DMA,
        ],
    )
    def decode(z_hbm, src_hbm, dst_hbm, rel_hbm, w_hbm, out_hbm,
               si_v, di_v, ri_v, sr0, dr0, rr0, sr1, dr1, rr1,
               t_v, oba, obb, sem0, sem1, sem_o):
        wid = lax.axis_index("s") * 2 + lax.axis_index("c")
        base0 = wid * epw
        iota = lax.iota(jnp.int32, LANES)

        pltpu.sync_copy(src_hbm.at[pl.ds(base0, epw)], si_v)
        pltpu.sync_copy(dst_hbm.at[pl.ds(base0, epw)], di_v)
        pltpu.sync_copy(rel_hbm.at[pl.ds(base0, epw)], ri_v)

        def row_copies(g, sr, dr, rr, sem):
            off = g * C
            return (
                pltpu.make_async_copy(z_hbm.at[si_v.at[pl.ds(off, C)]], sr, sem),
                pltpu.make_async_copy(z_hbm.at[di_v.at[pl.ds(off, C)]], dr, sem),
                pltpu.make_async_copy(w_hbm.at[ri_v.at[pl.ds(off, C)]], rr, sem),
            )

        def issue(g, sr, dr, rr, sem):
            for cp in row_copies(g, sr, dr, rr, sem):
                cp.start()

        def wait(g, sr, dr, rr, sem):
            for cp in row_copies(g, sr, dr, rr, sem):
                cp.wait()

        def mul2(sv, dv, wv):
            s0, s1 = plsc.unpack(plsc.bitcast(sv, jnp.bfloat16),
                                 format=plsc.PackFormat.INTERLEAVED)
            d0, d1 = plsc.unpack(plsc.bitcast(dv, jnp.bfloat16),
                                 format=plsc.PackFormat.INTERLEAVED)
            w0, w1 = plsc.unpack(plsc.bitcast(wv, jnp.bfloat16),
                                 format=plsc.PackFormat.INTERLEAVED)
            return s0 * d0 * w0 + s1 * d1 * w1

        def out_copy(g, ob):
            return pltpu.make_async_copy(
                ob, out_hbm.at[pl.ds(base0 + g * C, C)], sem_o)

        def score_chunk(g, sr, dr, rr, ob):
            def group_body(grp, carry):
                gb = grp * LANES

                for k in range(LANES):
                    acc = mul2(sr[gb + k, pl.ds(0, LANES)],
                               dr[gb + k, pl.ds(0, LANES)],
                               rr[gb + k, pl.ds(0, LANES)])
                    for j in range(1, DW // LANES):
                        acc = acc + mul2(sr[gb + k, pl.ds(j * LANES, LANES)],
                                         dr[gb + k, pl.ds(j * LANES, LANES)],
                                         rr[gb + k, pl.ds(j * LANES, LANES)])
                    t_v[k, :] = acc

                cols = [plsc.load_gather(
                            t_v, [iota, jnp.full((LANES,), k, jnp.int32)])
                        for k in range(LANES)]
                while len(cols) > 1:
                    cols = [a + b for a, b in zip(cols[0::2], cols[1::2])]
                s = 1.0 / (1.0 + jnp.exp(-cols[0]))
                ob[pl.ds(gb, LANES)] = s
                return carry

            lax.fori_loop(0, C // LANES, group_body, 0)
            out_copy(g, ob).start()

        bufs = ((sr0, dr0, rr0, sem0), (sr1, dr1, rr1, sem1))
        obs = (oba, obb)
        issue(0, *bufs[0])

        def pair_body(i, carry):
            g = 2 * i
            for p in range(2):
                gc = g + p
                issue(gc + 1, *bufs[1 - p])
                wait(gc, *bufs[p])

                @pl.when(gc >= 2)
                def _():
                    out_copy(gc - 2, obs[p]).wait()

                score_chunk(gc, *bufs[p][:3], obs[p])
            return carry

        lax.fori_loop(0, (n_chunks - 1) // 2, pair_body, 0)
        last = n_chunks - 1
        wait(last, *bufs[last % 2])
        out_copy(last - 2, obs[last % 2]).wait()
        score_chunk(last, *bufs[last % 2][:3], obs[last % 2])
        out_copy(n_chunks - 2, obs[(n_chunks - 2) % 2]).wait()
        out_copy(last, obs[last % 2]).wait()

    return decode(zp, src_idx, dst_idx, rel_idx, wp)


def kernel(z, edge_index, edge_type, weight):
    n_edges = edge_index.shape[1]
    src_idx = edge_index[0]
    dst_idx = edge_index[1]
    return _sc_decode(_pack_bf16(z), src_idx, dst_idx, edge_type,
                      _pack_bf16(weight), n_edges)


# speedup vs baseline: 9.3432x; 1.0674x over previous
import functools

import jax
import jax.numpy as jnp
from jax import lax
from jax.experimental import pallas as pl
from jax.experimental.pallas import tpu as pltpu
from jax.experimental.pallas import tpu_sc as plsc

D = 128
DW = D // 2
LANES = 16
NW = 32
C = 80


def _pack_bf16(a):
    n = a.shape[0]
    return lax.bitcast_convert_type(
        a.astype(jnp.bfloat16).reshape(n, DW, 2), jnp.float32)


def _sc_decode(zp, src_idx, dst_idx, rel_idx, wp, n_edges):
    epw = n_edges // NW
    n_chunks = epw // C

    mesh = plsc.VectorSubcoreMesh(core_axis_name="c", subcore_axis_name="s")

    @functools.partial(
        pl.kernel,
        out_type=jax.ShapeDtypeStruct((n_edges,), jnp.float32),
        mesh=mesh,
        compiler_params=pltpu.CompilerParams(needs_layout_passes=False,
                                             use_tc_tiling_on_sc=False),
        scratch_types=[
            pltpu.VMEM((epw,), jnp.int32),
            pltpu.VMEM((epw,), jnp.int32),
            pltpu.VMEM((epw,), jnp.int32),
            pltpu.VMEM((C, DW), jnp.float32),
            pltpu.VMEM((C, DW), jnp.float32),
            pltpu.VMEM((C, DW), jnp.float32),
            pltpu.VMEM((C, DW), jnp.float32),
            pltpu.VMEM((C, DW), jnp.float32),
            pltpu.VMEM((C, DW), jnp.float32),
            pltpu.VMEM((LANES, LANES), jnp.float32),
            pltpu.VMEM((C,), jnp.float32),
            pltpu.VMEM((C,), jnp.float32),
            pltpu.SemaphoreType.DMA,
            pltpu.SemaphoreType.DMA,
            pltpu.SemaphoreType.DMA,
        ],
    )
    def decode(z_hbm, src_hbm, dst_hbm, rel_hbm, w_hbm, out_hbm,
               si_v, di_v, ri_v, sr0, dr0, rr0, sr1, dr1, rr1,
               t_v, oba, obb, sem0, sem1, sem_o):
        wid = lax.axis_index("s") * 2 + lax.axis_index("c")
        base0 = wid * epw
        iota = lax.iota(jnp.int32, LANES)

        pltpu.sync_copy(src_hbm.at[pl.ds(base0, epw)], si_v)
        pltpu.sync_copy(dst_hbm.at[pl.ds(base0, epw)], di_v)
        pltpu.sync_copy(rel_hbm.at[pl.ds(base0, epw)], ri_v)

        def row_copies(g, sr, dr, rr, sem):
            off = g * C
            return (
                pltpu.make_async_copy(z_hbm.at[si_v.at[pl.ds(off, C)]], sr, sem),
                pltpu.make_async_copy(z_hbm.at[di_v.at[pl.ds(off, C)]], dr, sem),
                pltpu.make_async_copy(w_hbm.at[ri_v.at[pl.ds(off, C)]], rr, sem),
            )

        def issue(g, sr, dr, rr, sem):
            for cp in row_copies(g, sr, dr, rr, sem):
                cp.start()

        def wait(g, sr, dr, rr, sem):
            for cp in row_copies(g, sr, dr, rr, sem):
                cp.wait()

        def mul2(sv, dv, wv):
            p = (plsc.bitcast(sv, jnp.bfloat16)
                 * plsc.bitcast(dv, jnp.bfloat16)
                 * plsc.bitcast(wv, jnp.bfloat16))
            p0, p1 = plsc.unpack(p, format=plsc.PackFormat.INTERLEAVED)
            return p0 + p1

        def out_copy(g, ob):
            return pltpu.make_async_copy(
                ob, out_hbm.at[pl.ds(base0 + g * C, C)], sem_o)

        def score_chunk(g, sr, dr, rr, ob):
            def group_body(grp, carry):
                gb = grp * LANES

                for k in range(LANES):
                    acc = mul2(sr[gb + k, pl.ds(0, LANES)],
                               dr[gb + k, pl.ds(0, LANES)],
                               rr[gb + k, pl.ds(0, LANES)])
                    for j in range(1, DW // LANES):
                        acc = acc + mul2(sr[gb + k, pl.ds(j * LANES, LANES)],
                                         dr[gb + k, pl.ds(j * LANES, LANES)],
                                         rr[gb + k, pl.ds(j * LANES, LANES)])
                    t_v[k, :] = acc

                cols = [plsc.load_gather(
                            t_v, [iota, jnp.full((LANES,), k, jnp.int32)])
                        for k in range(LANES)]
                while len(cols) > 1:
                    cols = [a + b for a, b in zip(cols[0::2], cols[1::2])]
                s = 1.0 / (1.0 + jnp.exp(-cols[0]))
                ob[pl.ds(gb, LANES)] = s
                return carry

            lax.fori_loop(0, C // LANES, group_body, 0)
            out_copy(g, ob).start()

        bufs = ((sr0, dr0, rr0, sem0), (sr1, dr1, rr1, sem1))
        obs = (oba, obb)
        issue(0, *bufs[0])

        def pair_body(i, carry):
            g = 2 * i
            for p in range(2):
                gc = g + p
                issue(gc + 1, *bufs[1 - p])
                wait(gc, *bufs[p])

                @pl.when(gc >= 2)
                def _():
                    out_copy(gc - 2, obs[p]).wait()

                score_chunk(gc, *bufs[p][:3], obs[p])
            return carry

        lax.fori_loop(0, (n_chunks - 1) // 2, pair_body, 0)
        last = n_chunks - 1
        wait(last, *bufs[last % 2])
        out_copy(last - 2, obs[last % 2]).wait()
        score_chunk(last, *bufs[last % 2][:3], obs[last % 2])
        out_copy(n_chunks - 2, obs[(n_chunks - 2) % 2]).wait()
        out_copy(last, obs[last % 2]).wait()

    return decode(zp, src_idx, dst_idx, rel_idx, wp)


def kernel(z, edge_index, edge_type, weight):
    n_edges = edge_index.shape[1]
    src_idx = edge_index[0]
    dst_idx = edge_index[1]
    return _sc_decode(_pack_bf16(z), src_idx, dst_idx, edge_type,
                      _pack_bf16(weight), n_edges)


# speedup vs baseline: 11.2362x; 1.2026x over previous
import functools

import jax
import jax.numpy as jnp
from jax import lax
from jax.experimental import pallas as pl
from jax.experimental.pallas import tpu as pltpu
from jax.experimental.pallas import tpu_sc as plsc

D = 128
DW = D // 2
LANES = 16
NW = 32
C = 80


def _pack_bf16(a):
    n = a.shape[0]
    return lax.bitcast_convert_type(
        a.astype(jnp.bfloat16).reshape(n, DW, 2), jnp.float32)


def _sc_decode(zp, src_idx, dst_idx, rel_idx, wp, n_edges):
    epw = n_edges // NW
    n_chunks = epw // C

    mesh = plsc.VectorSubcoreMesh(core_axis_name="c", subcore_axis_name="s")

    @functools.partial(
        pl.kernel,
        out_type=jax.ShapeDtypeStruct((n_edges,), jnp.float32),
        mesh=mesh,
        compiler_params=pltpu.CompilerParams(needs_layout_passes=False,
                                             use_tc_tiling_on_sc=False),
        scratch_types=[
            pltpu.VMEM((epw,), jnp.int32),
            pltpu.VMEM((epw,), jnp.int32),
            pltpu.VMEM((epw,), jnp.int32),
            pltpu.VMEM((C, DW), jnp.float32),
            pltpu.VMEM((C, DW), jnp.float32),
            pltpu.VMEM((C, DW), jnp.float32),
            pltpu.VMEM((C, DW), jnp.float32),
            pltpu.VMEM((C, DW), jnp.float32),
            pltpu.VMEM((C, DW), jnp.float32),
            pltpu.VMEM((LANES, LANES), jnp.float32),
            pltpu.VMEM((C,), jnp.float32),
            pltpu.VMEM((C,), jnp.float32),
            pltpu.SemaphoreType.DMA,
            pltpu.SemaphoreType.DMA,
            pltpu.SemaphoreType.DMA,
        ],
    )
    def decode(z_hbm, src_hbm, dst_hbm, rel_hbm, w_hbm, out_hbm,
               si_v, di_v, ri_v, sr0, dr0, rr0, sr1, dr1, rr1,
               t_v, oba, obb, sem0, sem1, sem_o):
        wid = lax.axis_index("s") * 2 + lax.axis_index("c")
        base0 = wid * epw
        iota = lax.iota(jnp.int32, LANES)

        pltpu.sync_copy(src_hbm.at[pl.ds(base0, epw)], si_v)
        pltpu.sync_copy(dst_hbm.at[pl.ds(base0, epw)], di_v)
        pltpu.sync_copy(rel_hbm.at[pl.ds(base0, epw)], ri_v)

        def row_copies(g, sr, dr, rr, sem):
            off = g * C
            return (
                pltpu.make_async_copy(z_hbm.at[si_v.at[pl.ds(off, C)]], sr, sem),
                pltpu.make_async_copy(z_hbm.at[di_v.at[pl.ds(off, C)]], dr, sem),
                pltpu.make_async_copy(w_hbm.at[ri_v.at[pl.ds(off, C)]], rr, sem),
            )

        def issue(g, sr, dr, rr, sem):
            for cp in row_copies(g, sr, dr, rr, sem):
                cp.start()

        def wait(g, sr, dr, rr, sem):
            for cp in row_copies(g, sr, dr, rr, sem):
                cp.wait()

        def edge_loads(sr, dr, rr, gb, k):
            return [(sr[gb + k, pl.ds(j * LANES, LANES)],
                     dr[gb + k, pl.ds(j * LANES, LANES)],
                     rr[gb + k, pl.ds(j * LANES, LANES)])
                    for j in range(DW // LANES)]

        def edge_score(loaded):
            halves = []
            for sv, dv, wv in loaded:
                p = (plsc.bitcast(sv, jnp.bfloat16)
                     * plsc.bitcast(dv, jnp.bfloat16)
                     * plsc.bitcast(wv, jnp.bfloat16))
                p0, p1 = plsc.unpack(p, format=plsc.PackFormat.INTERLEAVED)
                halves += [p0, p1]
            while len(halves) > 1:
                halves = [a + b for a, b in zip(halves[0::2], halves[1::2])]
            return halves[0]

        def out_copy(g, ob):
            return pltpu.make_async_copy(
                ob, out_hbm.at[pl.ds(base0 + g * C, C)], sem_o)

        def score_chunk(g, sr, dr, rr, ob):
            def group_body(grp, carry):
                gb = grp * LANES

                cur = edge_loads(sr, dr, rr, gb, 0)
                for k in range(LANES):
                    nxt = (edge_loads(sr, dr, rr, gb, k + 1)
                           if k + 1 < LANES else None)
                    t_v[k, :] = edge_score(cur)
                    cur = nxt

                cols = [plsc.load_gather(
                            t_v, [iota, jnp.full((LANES,), k, jnp.int32)])
                        for k in range(LANES)]
                while len(cols) > 1:
                    cols = [a + b for a, b in zip(cols[0::2], cols[1::2])]
                s = 1.0 / (1.0 + jnp.exp(-cols[0]))
                ob[pl.ds(gb, LANES)] = s
                return carry

            lax.fori_loop(0, C // LANES, group_body, 0)
            out_copy(g, ob).start()

        bufs = ((sr0, dr0, rr0, sem0), (sr1, dr1, rr1, sem1))
        obs = (oba, obb)
        issue(0, *bufs[0])

        def pair_body(i, carry):
            g = 2 * i
            for p in range(2):
                gc = g + p
                issue(gc + 1, *bufs[1 - p])
                wait(gc, *bufs[p])

                @pl.when(gc >= 2)
                def _():
                    out_copy(gc - 2, obs[p]).wait()

                score_chunk(gc, *bufs[p][:3], obs[p])
            return carry

        lax.fori_loop(0, (n_chunks - 1) // 2, pair_body, 0)
        last = n_chunks - 1
        wait(last, *bufs[last % 2])
        out_copy(last - 2, obs[last % 2]).wait()
        score_chunk(last, *bufs[last % 2][:3], obs[last % 2])
        out_copy(n_chunks - 2, obs[(n_chunks - 2) % 2]).wait()
        out_copy(last, obs[last % 2]).wait()

    return decode(zp, src_idx, dst_idx, rel_idx, wp)


def kernel(z, edge_index, edge_type, weight):
    n_edges = edge_index.shape[1]
    src_idx = edge_index[0]
    dst_idx = edge_index[1]
    return _sc_decode(_pack_bf16(z), src_idx, dst_idx, edge_type,
                      _pack_bf16(weight), n_edges)


# speedup vs baseline: 12.5266x; 1.1148x over previous
import functools

import jax
import jax.numpy as jnp
from jax import lax
from jax.experimental import pallas as pl
from jax.experimental.pallas import tpu as pltpu
from jax.experimental.pallas import tpu_sc as plsc

D = 128
DW = D // 2
LANES = 16
NW = 32
C = 80


def _pack_bf16(a):
    n = a.shape[0]
    return lax.bitcast_convert_type(
        a.astype(jnp.bfloat16).reshape(n, DW, 2), jnp.float32)


def _sc_decode(zp, src_idx, dst_idx, rel_idx, wp, n_edges):
    epw = n_edges // NW
    n_chunks = epw // C

    mesh = plsc.VectorSubcoreMesh(core_axis_name="c", subcore_axis_name="s")

    @functools.partial(
        pl.kernel,
        out_type=jax.ShapeDtypeStruct((n_edges,), jnp.float32),
        mesh=mesh,
        compiler_params=pltpu.CompilerParams(needs_layout_passes=False,
                                             use_tc_tiling_on_sc=False),
        scratch_types=[
            pltpu.VMEM((epw,), jnp.int32),
            pltpu.VMEM((epw,), jnp.int32),
            pltpu.VMEM((epw,), jnp.int32),
            pltpu.VMEM((C, DW), jnp.float32),
            pltpu.VMEM((C, DW), jnp.float32),
            pltpu.VMEM((C, DW), jnp.float32),
            pltpu.VMEM((C, DW), jnp.float32),
            pltpu.VMEM((C, DW), jnp.float32),
            pltpu.VMEM((C, DW), jnp.float32),
            pltpu.VMEM((C, DW), jnp.float32),
            pltpu.VMEM((C, DW), jnp.float32),
            pltpu.VMEM((C, DW), jnp.float32),
            pltpu.VMEM((LANES, LANES), jnp.float32),
            pltpu.VMEM((C,), jnp.float32),
            pltpu.VMEM((C,), jnp.float32),
            pltpu.VMEM((C,), jnp.float32),
            pltpu.SemaphoreType.DMA,
            pltpu.SemaphoreType.DMA,
            pltpu.SemaphoreType.DMA,
            pltpu.SemaphoreType.DMA,
        ],
    )
    def decode(z_hbm, src_hbm, dst_hbm, rel_hbm, w_hbm, out_hbm,
               si_v, di_v, ri_v, sr0, dr0, rr0, sr1, dr1, rr1,
               sr2, dr2, rr2, t_v, ob0, ob1, ob2, sem0, sem1, sem2, sem_o):
        wid = lax.axis_index("s") * 2 + lax.axis_index("c")
        base0 = wid * epw
        iota = lax.iota(jnp.int32, LANES)

        pltpu.sync_copy(src_hbm.at[pl.ds(base0, epw)], si_v)
        pltpu.sync_copy(dst_hbm.at[pl.ds(base0, epw)], di_v)
        pltpu.sync_copy(rel_hbm.at[pl.ds(base0, epw)], ri_v)

        def row_copies(g, sr, dr, rr, sem):
            off = g * C
            return (
                pltpu.make_async_copy(z_hbm.at[si_v.at[pl.ds(off, C)]], sr, sem),
                pltpu.make_async_copy(z_hbm.at[di_v.at[pl.ds(off, C)]], dr, sem),
                pltpu.make_async_copy(w_hbm.at[ri_v.at[pl.ds(off, C)]], rr, sem),
            )

        def issue(g, sr, dr, rr, sem):
            for cp in row_copies(g, sr, dr, rr, sem):
                cp.start()

        def wait(g, sr, dr, rr, sem):
            for cp in row_copies(g, sr, dr, rr, sem):
                cp.wait()

        def edge_loads(sr, dr, rr, gb, k):
            return [(sr[gb + k, pl.ds(j * LANES, LANES)],
                     dr[gb + k, pl.ds(j * LANES, LANES)],
                     rr[gb + k, pl.ds(j * LANES, LANES)])
                    for j in range(DW // LANES)]

        def edge_score(loaded):
            halves = []
            for sv, dv, wv in loaded:
                p = (plsc.bitcast(sv, jnp.bfloat16)
                     * plsc.bitcast(dv, jnp.bfloat16)
                     * plsc.bitcast(wv, jnp.bfloat16))
                p0, p1 = plsc.unpack(p, format=plsc.PackFormat.INTERLEAVED)
                halves += [p0, p1]
            while len(halves) > 1:
                halves = [a + b for a, b in zip(halves[0::2], halves[1::2])]
            return halves[0]

        def out_copy(g, ob):
            return pltpu.make_async_copy(
                ob, out_hbm.at[pl.ds(base0 + g * C, C)], sem_o)

        def score_chunk(g, sr, dr, rr, ob):
            def group_body(grp, carry):
                gb = grp * LANES

                cur = edge_loads(sr, dr, rr, gb, 0)
                for k in range(LANES):
                    nxt = (edge_loads(sr, dr, rr, gb, k + 1)
                           if k + 1 < LANES else None)
                    t_v[k, :] = edge_score(cur)
                    cur = nxt

                cols = [plsc.load_gather(
                            t_v, [iota, jnp.full((LANES,), k, jnp.int32)])
                        for k in range(LANES)]
                while len(cols) > 1:
                    cols = [a + b for a, b in zip(cols[0::2], cols[1::2])]
                s = 1.0 / (1.0 + jnp.exp(-cols[0]))
                ob[pl.ds(gb, LANES)] = s
                return carry

            lax.fori_loop(0, C // LANES, group_body, 0)
            out_copy(g, ob).start()

        bufs = ((sr0, dr0, rr0, sem0), (sr1, dr1, rr1, sem1),
                (sr2, dr2, rr2, sem2))
        obs = (ob0, ob1, ob2)
        issue(0, *bufs[0])
        issue(1, *bufs[1])

        def triple_body(i, carry):
            g = 3 * i
            for p in range(3):
                gc = g + p
                issue(gc + 2, *bufs[(p + 2) % 3])
                wait(gc, *bufs[p])

                @pl.when(gc >= 3)
                def _():
                    out_copy(gc - 3, obs[p]).wait()

                score_chunk(gc, *bufs[p][:3], obs[p])
            return carry

        lax.fori_loop(0, (n_chunks - 2) // 3, triple_body, 0)
        for g in range(n_chunks - 2, n_chunks):
            wait(g, *bufs[g % 3])
            out_copy(g - 3, obs[g % 3]).wait()
            score_chunk(g, *bufs[g % 3][:3], obs[g % 3])
        for g in range(n_chunks - 3, n_chunks):
            out_copy(g, obs[g % 3]).wait()

    return decode(zp, src_idx, dst_idx, rel_idx, wp)


def kernel(z, edge_index, edge_type, weight):
    n_edges = edge_index.shape[1]
    src_idx = edge_index[0]
    dst_idx = edge_index[1]
    return _sc_decode(_pack_bf16(z), src_idx, dst_idx, edge_type,
                      _pack_bf16(weight), n_edges)
